# baseline (speedup 1.0000x reference)
import functools

import jax
import jax.numpy as jnp
from jax import lax
from jax.experimental import pallas as pl
from jax.experimental.pallas import tpu as pltpu
from jax.experimental.pallas import tpu_sc as plsc

EPS = 1e-6
LANES = 16
CHUNK = 128


def _lane_sum(x, perms):
    for p in perms:
        x = x + x.at[p].get(mode="promise_in_bounds")
    return x


def _rsqrt(x):
    xi = lax.bitcast_convert_type(x, jnp.int32)
    yi = jnp.int32(0x5F3759DF) - (xi >> 1)
    y = lax.bitcast_convert_type(yi, jnp.float32)
    for _ in range(3):
        y = y * (1.5 - 0.5 * x * y * y)
    return y


def kernel(src, seg, table, gamma, beta):
    del seg
    B, L = src.shape
    V, E = table.shape
    n_vec = E // LANES
    N = B * L

    info = plsc.get_sparse_core_info()
    NC, NS = info.num_cores, info.num_subcores
    NW = NC * NS
    per_w = N // NW
    n_chunks = per_w // CHUNK
    assert per_w * NW == N and n_chunks * CHUNK == per_w

    idx = src.reshape(NW, n_chunks, CHUNK)
    mesh = plsc.VectorSubcoreMesh(core_axis_name="c", subcore_axis_name="s")

    @functools.partial(
        pl.kernel,
        mesh=mesh,
        out_type=jax.ShapeDtypeStruct((N, E), jnp.float32),
        compiler_params=pltpu.CompilerParams(use_tc_tiling_on_sc=False),
        scratch_types=[
            pltpu.VMEM((n_chunks, CHUNK), jnp.int32),
            pltpu.VMEM((CHUNK, E), jnp.float32),
            pltpu.VMEM((CHUNK, E), jnp.float32),
            pltpu.VMEM((E,), jnp.float32),
            pltpu.VMEM((E,), jnp.float32),
            pltpu.SemaphoreType.DMA,
        ],
    )
    def emb_ln(table_hbm, idx_hbm, gamma_hbm, beta_hbm, out_hbm,
               idx_v, rows_v, out_v, gamma_v, beta_v, sem):
        wid = lax.axis_index("s") * NC + lax.axis_index("c")
        base = wid * per_w
        pltpu.sync_copy(idx_hbm.at[wid], idx_v)
        pltpu.sync_copy(gamma_hbm, gamma_v)
        pltpu.sync_copy(beta_hbm, beta_v)
        g = [gamma_v[pl.ds(j * LANES, LANES)] for j in range(n_vec)]
        bta = [beta_v[pl.ds(j * LANES, LANES)] for j in range(n_vec)]
        inv_e = jnp.float32(1.0 / E)
        lane = lax.iota(jnp.int32, LANES)
        perms = [jnp.bitwise_xor(lane, jnp.int32(1 << k)) for k in range(4)]

        def chunk_body(c, _):
            pltpu.async_copy(table_hbm.at[idx_v.at[c]], rows_v, sem).wait()

            def row_body(r, _):
                x = [rows_v[r, pl.ds(j * LANES, LANES)] for j in range(n_vec)]
                s = x[0]
                q = x[0] * x[0]
                for j in range(1, n_vec):
                    s = s + x[j]
                    q = q + x[j] * x[j]
                tot = _lane_sum(s, perms)
                qtot = _lane_sum(q, perms)
                mean = tot * inv_e
                var = qtot * inv_e - mean * mean
                inv = _rsqrt(var + EPS)
                for j in range(n_vec):
                    out_v[r, pl.ds(j * LANES, LANES)] = (
                        (x[j] - mean) * inv * g[j] + bta[j])
                return 0

            lax.fori_loop(0, CHUNK, row_body, 0, unroll=2)
            pltpu.sync_copy(out_v, out_hbm.at[pl.ds(base + c * CHUNK, CHUNK)])
            return 0

        lax.fori_loop(0, n_chunks, chunk_body, 0)

    out = emb_ln(table, idx, gamma, beta)
    return out.reshape(B, L, E)


# speedup vs baseline: 1.0114x; 1.0114x over previous
import functools

import jax
import jax.numpy as jnp
from jax import lax
from jax.experimental import pallas as pl
from jax.experimental.pallas import tpu as pltpu
from jax.experimental.pallas import tpu_sc as plsc

EPS = 1e-6
LANES = 16
CHUNK = 128


def _rsqrt(x):
    xi = lax.bitcast_convert_type(x, jnp.int32)
    yi = jnp.int32(0x5F3759DF) - (xi >> 1)
    y = lax.bitcast_convert_type(yi, jnp.float32)
    for _ in range(2):
        y = y * (1.5 - 0.5 * x * y * y)
    return y


def kernel(src, seg, table, gamma, beta):
    del seg
    B, L = src.shape
    V, E = table.shape
    n_vec = E // LANES
    N = B * L

    info = plsc.get_sparse_core_info()
    NC, NS = info.num_cores, info.num_subcores
    NW = NC * NS
    per_w = N // NW
    n_chunks = per_w // CHUNK
    assert per_w * NW == N and n_chunks * CHUNK == per_w
    assert n_chunks >= 4 and n_chunks % 2 == 0
    assert E == 64 and V % 2 == 0 and CHUNK % (2 * LANES) == 0

    src_flat = src.reshape(NW, n_chunks, CHUNK)
    idx2 = src_flat >> 1
    halfoff = (src_flat & 1) * E
    table2 = table.reshape(V // 2, 2 * E)
    mesh = plsc.VectorSubcoreMesh(core_axis_name="c", subcore_axis_name="s")

    @functools.partial(
        pl.kernel,
        mesh=mesh,
        out_type=jax.ShapeDtypeStruct((N // 2, 2 * E), jnp.float32),
        compiler_params=pltpu.CompilerParams(needs_layout_passes=False),
        scratch_types=[
            pltpu.VMEM((n_chunks, CHUNK), jnp.int32),
            pltpu.VMEM((n_chunks, CHUNK), jnp.int32),
            pltpu.VMEM((CHUNK, 2 * E), jnp.float32),
            pltpu.VMEM((CHUNK, 2 * E), jnp.float32),
            pltpu.VMEM((CHUNK // 2, 2 * E), jnp.float32),
            pltpu.VMEM((CHUNK // 2, 2 * E), jnp.float32),
            pltpu.VMEM((LANES, LANES), jnp.float32),
            pltpu.VMEM((LANES, LANES), jnp.float32),
            pltpu.VMEM((E,), jnp.float32),
            pltpu.VMEM((E,), jnp.float32),
            pltpu.SemaphoreType.DMA,
            pltpu.SemaphoreType.DMA,
            pltpu.SemaphoreType.DMA,
            pltpu.SemaphoreType.DMA,
        ],
    )
    def emb_ln(table_hbm, idx_hbm, half_hbm, gamma_hbm, beta_hbm, out_hbm,
               idx_v, half_v, rows0, rows1, outv0, outv1, sbuf, qbuf,
               gamma_v, beta_v, gsem0, gsem1, osem0, osem1):
        rows_vs = (rows0, rows1)
        out_vs = (outv0, outv1)
        gsems = (gsem0, gsem1)
        osems = (osem0, osem1)
        och = CHUNK // 2

        wid = lax.axis_index("s") * NC + lax.axis_index("c")
        base = wid * per_w
        obase = base // 2
        pltpu.sync_copy(idx_hbm.at[wid], idx_v)
        pltpu.sync_copy(half_hbm.at[wid], half_v)
        pltpu.sync_copy(gamma_hbm, gamma_v)
        pltpu.sync_copy(beta_hbm, beta_v)
        g = [gamma_v[pl.ds(j * LANES, LANES)] for j in range(n_vec)]
        bta = [beta_v[pl.ds(j * LANES, LANES)] for j in range(n_vec)]
        inv_e = jnp.float32(1.0 / E)
        lane = lax.iota(jnp.int32, LANES)

        def gather_start(c, b):
            pltpu.async_copy(table_hbm.at[idx_v.at[c]], rows_vs[b], gsems[b])

        def gather_wait(c, b):
            pltpu.make_async_copy(
                table_hbm.at[idx_v.at[c]], rows_vs[b], gsems[b]).wait()

        def out_start(c, b):
            start = pl.multiple_of(obase + c * och, 8)
            pltpu.async_copy(
                out_vs[b], out_hbm.at[pl.ds(start, och)], osems[b])

        def out_wait(c, b):
            start = pl.multiple_of(obase + c * och, 8)
            pltpu.make_async_copy(
                out_vs[b], out_hbm.at[pl.ds(start, och)], osems[b]).wait()

        def compute(c, b):
            rows_v = rows_vs[b]
            out_v = out_vs[b]

            def group_body(gi, _):
                g0 = gi * LANES
                hv = half_v[c, pl.ds(g0, LANES)]
                offs = []
                for rr in range(LANES):
                    r = g0 + rr
                    off = hv[rr]
                    offs.append(off)
                    x = [rows_v[r, pl.ds(off + j * LANES, LANES)]
                         for j in range(n_vec)]
                    s = (x[0] + x[1]) + (x[2] + x[3])
                    q = (x[0] * x[0] + x[1] * x[1]) + (
                        x[2] * x[2] + x[3] * x[3])
                    sbuf[rr, pl.ds(0, LANES)] = s
                    qbuf[rr, pl.ds(0, LANES)] = q
                svs = [plsc.load_gather(
                    sbuf, [lane, jnp.full((LANES,), l, jnp.int32)])
                    for l in range(LANES)]
                qvs = [plsc.load_gather(
                    qbuf, [lane, jnp.full((LANES,), l, jnp.int32)])
                    for l in range(LANES)]
                while len(svs) > 1:
                    svs = [a + c2 for a, c2 in zip(svs[::2], svs[1::2])]
                while len(qvs) > 1:
                    qvs = [a + c2 for a, c2 in zip(qvs[::2], qvs[1::2])]
                mean = svs[0] * inv_e
                var = qvs[0] * inv_e - mean * mean
                inv = _rsqrt(var + EPS)
                for rr in range(LANES):
                    r = g0 + rr
                    off = offs[rr]
                    sel = jnp.full((LANES,), rr, jnp.int32)
                    m_r = mean.at[sel].get(mode="promise_in_bounds")
                    i_r = inv.at[sel].get(mode="promise_in_bounds")
                    out_r = (g0 + rr) // 2
                    out_c = (rr % 2) * E
                    for j in range(n_vec):
                        xj = rows_v[r, pl.ds(off + j * LANES, LANES)]
                        out_v[out_r, pl.ds(out_c + j * LANES, LANES)] = (
                            (xj - m_r) * (i_r * g[j]) + bta[j])
                return 0

            lax.fori_loop(0, CHUNK // LANES, group_body, 0)

        def stage(c0, b):
            c = c0 + b
            gather_wait(c, b)
            pl.when(c >= 2)(lambda: out_wait(c - 2, b))
            compute(c, b)
            out_start(c, b)
            pl.when(c + 2 < n_chunks)(lambda: gather_start(c + 2, b))

        gather_start(0, 0)
        gather_start(1, 1)

        def loop_body(i, _):
            c0 = 2 * i
            stage(c0, 0)
            stage(c0, 1)
            return 0

        lax.fori_loop(0, n_chunks // 2, loop_body, 0)
        out_wait(n_chunks - 2, 0)
        out_wait(n_chunks - 1, 1)

    out = emb_ln(table2, idx2, halfoff, gamma, beta)
    return out.reshape(B, L, E)
